# Initial kernel scaffold; baseline (speedup 1.0000x reference)
import jax
import jax.numpy as jnp
from jax.experimental import pallas as pl


def kernel(in_feat, edge_index, W1, b1, W2, b2):
    raise NotImplementedError("write your pallas kernel here")


# speedup vs baseline: 7.2737x; 7.2737x over previous
import functools

import jax
import jax.numpy as jnp
from jax import lax
from jax.experimental import pallas as pl
from jax.experimental.pallas import tpu as pltpu
from jax.experimental.pallas import tpu_sc as plsc

N = 10000
E = 320000
NC, NS = 2, 16
NW = NC * NS
EPW = E // NW
CHUNK = 128
NCH = EPW // CHUNK
TAIL = EPW - NCH * CHUNK
RPS = 624
RTL = N - RPS * NS
BLK = 1000
G = N // BLK


@functools.lru_cache(maxsize=None)
def _make_prop(nphase, w):
  mesh = plsc.VectorSubcoreMesh(
      core_axis_name="c", subcore_axis_name="s",
      num_cores=NC, num_subcores=NS)

  @functools.partial(
      pl.kernel,
      out_type=[jax.ShapeDtypeStruct((NC, N, w), jnp.float32)
                for _ in range(nphase)],
      mesh=mesh,
      compiler_params=pltpu.CompilerParams(use_tc_tiling_on_sc=False),
      scratch_types=[
          pltpu.VMEM((NCH, CHUNK), jnp.int32),
          pltpu.VMEM((NCH, CHUNK), jnp.int32),
          pltpu.VMEM((TAIL,), jnp.int32),
          pltpu.VMEM((TAIL,), jnp.int32),
          pltpu.VMEM((2, CHUNK, w), jnp.float32),
          pltpu.VMEM((TAIL, w), jnp.float32),
          pltpu.VMEM_SHARED((N, w), jnp.float32),
          pltpu.SemaphoreType.DMA,
          pltpu.SemaphoreType.DMA,
      ],
  )
  def prop(*args):
    src_m, dst_m, src_t, dst_t = args[0:4]
    hns = args[4:4 + nphase]
    zeros = args[4 + nphase]
    outs = args[5 + nphase:5 + 2 * nphase]
    si, di, st, dt, rows, rt, acc, s0, s1 = args[5 + 2 * nphase:]
    c = lax.axis_index("c")
    s = lax.axis_index("s")
    wk = c * NS + s
    sems = (s0, s1)

    pltpu.sync_copy(src_m.at[wk], si)
    pltpu.sync_copy(dst_m.at[wk], di)
    pltpu.sync_copy(src_t.at[wk], st)
    pltpu.sync_copy(dst_t.at[wk], dt)

    for p in range(nphase):
      hn = hns[p]
      pltpu.sync_copy(zeros.at[pl.ds(s * RPS, RPS)],
                      acc.at[pl.ds(s * RPS, RPS)])

      @pl.when(s == NS - 1)
      def _():
        pltpu.sync_copy(zeros.at[pl.ds(NS * RPS, RTL)],
                        acc.at[pl.ds(NS * RPS, RTL)])

      pltpu.async_copy(hn.at[si.at[0]], rows.at[0], s0)
      pltpu.async_copy(hn.at[si.at[1]], rows.at[1], s1)
      plsc.subcore_barrier()

      def pair(jo, carry):
        for b in range(2):
          j = jo * 2 + b
          pltpu.make_async_copy(hn.at[si.at[j]], rows.at[b], sems[b]).wait()
          pltpu.sync_copy(rows.at[b], acc.at[di.at[j]], add=True)
          pltpu.async_copy(hn.at[si.at[j + 2]], rows.at[b], sems[b])
        return carry
      lax.fori_loop(0, NCH // 2 - 1, pair, 0)

      for j, b in ((NCH - 2, 0), (NCH - 1, 1)):
        pltpu.make_async_copy(hn.at[si.at[j]], rows.at[b], sems[b]).wait()
        pltpu.sync_copy(rows.at[b], acc.at[di.at[j]], add=True)

      pltpu.async_copy(hn.at[st], rt, s0).wait()
      pltpu.sync_copy(rt, acc.at[dt], add=True)

      plsc.subcore_barrier()
      pltpu.sync_copy(acc.at[pl.ds(s * RPS, RPS)],
                      outs[p].at[c, pl.ds(s * RPS, RPS)])

      @pl.when(s == NS - 1)
      def _():
        pltpu.sync_copy(acc.at[pl.ds(NS * RPS, RTL)],
                        outs[p].at[c, pl.ds(NS * RPS, RTL)])

  return prop


def _prop(idx, hns, w):
  src_m, dst_m, src_t, dst_t = idx
  zeros = jnp.zeros((N, w), jnp.float32)
  outs = _make_prop(len(hns), w)(src_m, dst_m, src_t, dst_t, *hns, zeros)
  return outs if isinstance(outs, (list, tuple)) else [outs]


def _row_spec(*dims):
  nd = len(dims)
  if nd == 2:
    return pl.BlockSpec((BLK, dims[1]), lambda i: (i, 0))
  return pl.BlockSpec((dims[0], BLK, dims[2]), lambda i: (0, i, 0))


def _full_spec(shape):
  nd = len(shape)
  return pl.BlockSpec(shape, lambda i: (0,) * nd)


def _norm_hn(degp, x):
  def body(degp_r, x_r, norm_o, *hn_o):
    d = degp_r[...]
    deg = d[0, :, 0:1] + d[1, :, 0:1]
    nrm = lax.rsqrt(jnp.maximum(deg, 1.0))
    norm_o[...] = nrm
    hn = x_r[...] * nrm
    for q in range(4):
      hn_o[q][...] = hn[:, 32 * q:32 * (q + 1)]
  return pl.pallas_call(
      body, grid=(G,),
      in_specs=[_row_spec(2, N, 8), _row_spec(N, 128)],
      out_specs=[_row_spec(N, 1)] + [_row_spec(N, 32)] * 4,
      out_shape=[jax.ShapeDtypeStruct((N, 1), jnp.float32)]
      + [jax.ShapeDtypeStruct((N, 32), jnp.float32)] * 4,
  )(degp, x)


def _combine_hop(parts, norm):
  def body(p0, p1, p2, p3, norm_r, ph_o, *hn_o):
    nrm = norm_r[...]
    cols = []
    for q, pr in enumerate((p0, p1, p2, p3)):
      pv = pr[...]
      col = (pv[0] + pv[1]) * nrm
      cols.append(col)
      hn_o[q][...] = col * nrm
    ph_o[...] = jnp.concatenate(cols, axis=1)
  return pl.pallas_call(
      body, grid=(G,),
      in_specs=[_row_spec(2, N, 32)] * 4 + [_row_spec(N, 1)],
      out_specs=[_row_spec(N, 128)] + [_row_spec(N, 32)] * 4,
      out_shape=[jax.ShapeDtypeStruct((N, 128), jnp.float32)]
      + [jax.ShapeDtypeStruct((N, 32), jnp.float32)] * 4,
  )(*parts, norm)


def _layer1(x, px, p2, norm, W1, b1, wcat):
  def body(x_r, px_r, q0, q1, q2, q3, norm_r, w1_r, b1_r, wc_r,
           m2_o, h3a_o, h3b_o):
    nrm = norm_r[...]
    p2x = jnp.concatenate(
        [(q[...][0] + q[...][1]) for q in (q0, q1, q2, q3)], axis=1) * nrm
    w1 = w1_r[...]
    h1 = (jnp.dot(x_r[...], w1[0:128], preferred_element_type=jnp.float32)
          + jnp.dot(px_r[...], w1[128:256], preferred_element_type=jnp.float32)
          + jnp.dot(p2x, w1[256:384], preferred_element_type=jnp.float32)
          + b1_r[...])
    h1 = jnp.maximum(h1, 0.0)
    m2 = jnp.dot(h1, wc_r[...], preferred_element_type=jnp.float32)
    m2_o[...] = m2
    h3a_o[...] = m2[:, 40:80] * nrm
    h3b_o[...] = m2[:, 80:120] * nrm
  return pl.pallas_call(
      body, grid=(G,),
      in_specs=[_row_spec(N, 128), _row_spec(N, 128)]
      + [_row_spec(2, N, 32)] * 4
      + [_row_spec(N, 1), _full_spec((384, 128)), _full_spec((1, 128)),
         _full_spec((128, 120))],
      out_specs=[_row_spec(N, 120), _row_spec(N, 40), _row_spec(N, 40)],
      out_shape=[jax.ShapeDtypeStruct((N, 120), jnp.float32),
                 jax.ShapeDtypeStruct((N, 40), jnp.float32),
                 jax.ShapeDtypeStruct((N, 40), jnp.float32)],
  )(x, px, *p2, norm, W1, b1, wcat)


def _combine_l2hop1(parts, norm):
  def body(pa, pb, norm_r, zb_o, hn4_o):
    nrm = norm_r[...]
    pav = pa[...]
    pbv = pb[...]
    zb_o[...] = (pav[0] + pav[1]) * nrm
    hn4_o[...] = (pbv[0] + pbv[1]) * nrm * nrm
  return pl.pallas_call(
      body, grid=(G,),
      in_specs=[_row_spec(2, N, 40)] * 2 + [_row_spec(N, 1)],
      out_specs=[_row_spec(N, 40)] * 2,
      out_shape=[jax.ShapeDtypeStruct((N, 40), jnp.float32)] * 2,
  )(*parts, norm)


def _finalize(m2, zb, p4, norm, b2):
  def body(m2_r, zb_r, p4_r, norm_r, b2_r, out_o):
    i = pl.program_id(0)
    pv = p4_r[...]
    wv = (pv[0] + pv[1]) * norm_r[...]
    h2 = jnp.maximum(m2_r[...][:, 0:40] + zb_r[...] + wv + b2_r[...], 0.0)
    part = jnp.sum(h2, axis=0, keepdims=True) * (1.0 / N)

    @pl.when(i == 0)
    def _():
      out_o[...] = part

    @pl.when(i != 0)
    def _():
      out_o[...] += part

  return pl.pallas_call(
      body, grid=(G,),
      in_specs=[_row_spec(N, 120), _row_spec(N, 40), _row_spec(2, N, 40),
                _row_spec(N, 1), _full_spec((1, 40))],
      out_specs=pl.BlockSpec((1, 40), lambda i: (0, 0)),
      out_shape=jax.ShapeDtypeStruct((1, 40), jnp.float32),
  )(m2, zb, p4, norm, b2)


def kernel(in_feat, edge_index, W1, b1, W2, b2):
  src = edge_index[0].reshape(NW, EPW)
  dst = edge_index[1].reshape(NW, EPW)
  idx = (src[:, :NCH * CHUNK].reshape(NW, NCH, CHUNK),
         dst[:, :NCH * CHUNK].reshape(NW, NCH, CHUNK),
         src[:, NCH * CHUNK:],
         dst[:, NCH * CHUNK:])

  degp = _prop(idx, [jnp.ones((N, 8), jnp.float32)], 8)[0]
  norm, *hn1 = _norm_hn(degp, in_feat)

  p1 = _prop(idx, hn1, 32)
  px, *hn2 = _combine_hop(p1, norm)
  p2 = _prop(idx, hn2, 32)

  wcat = jnp.concatenate([W2[0:128], W2[128:256], W2[256:384]], axis=1)
  m2, h3a, h3b = _layer1(in_feat, px, p2, norm, W1, b1.reshape(1, 128), wcat)

  p3 = _prop(idx, [h3a, h3b], 40)
  zb, hn4 = _combine_l2hop1(p3, norm)
  p4 = _prop(idx, [hn4], 40)

  return _finalize(m2, zb, p4[0], norm, b2.reshape(1, 40))


# speedup vs baseline: 8.9082x; 1.2247x over previous
import functools

import jax
import jax.numpy as jnp
from jax import lax
from jax.experimental import pallas as pl
from jax.experimental.pallas import tpu as pltpu
from jax.experimental.pallas import tpu_sc as plsc

N = 10000
E = 320000
NC, NS = 2, 16
NW = NC * NS
EPW = E // NW
CHUNK = 128
NCH = EPW // CHUNK
TAIL = EPW - NCH * CHUNK
RPS = 624
RTL = N - RPS * NS
BLK = 1000
G = N // BLK


@functools.lru_cache(maxsize=None)
def _make_prop(nphase, w):
  mesh = plsc.VectorSubcoreMesh(
      core_axis_name="c", subcore_axis_name="s",
      num_cores=NC, num_subcores=NS)

  @functools.partial(
      pl.kernel,
      out_type=[jax.ShapeDtypeStruct((NC, N, w), jnp.float32)
                for _ in range(nphase)],
      mesh=mesh,
      compiler_params=pltpu.CompilerParams(use_tc_tiling_on_sc=False),
      scratch_types=[
          pltpu.VMEM((NCH, CHUNK), jnp.int32),
          pltpu.VMEM((NCH, CHUNK), jnp.int32),
          pltpu.VMEM((TAIL,), jnp.int32),
          pltpu.VMEM((TAIL,), jnp.int32),
          pltpu.VMEM((2, CHUNK, w), jnp.float32),
          pltpu.VMEM((TAIL, w), jnp.float32),
          pltpu.VMEM_SHARED((N, w), jnp.float32),
          pltpu.SemaphoreType.DMA,
          pltpu.SemaphoreType.DMA,
      ],
  )
  def prop(*args):
    src_m, dst_m, src_t, dst_t = args[0:4]
    hns = args[4:4 + nphase]
    zeros = args[4 + nphase]
    outs = args[5 + nphase:5 + 2 * nphase]
    si, di, st, dt, rows, rt, acc, s0, s1 = args[5 + 2 * nphase:]
    c = lax.axis_index("c")
    s = lax.axis_index("s")
    wk = c * NS + s
    sems = (s0, s1)

    pltpu.sync_copy(src_m.at[wk], si)
    pltpu.sync_copy(dst_m.at[wk], di)
    pltpu.sync_copy(src_t.at[wk], st)
    pltpu.sync_copy(dst_t.at[wk], dt)

    for p in range(nphase):
      hn = hns[p]
      pltpu.sync_copy(zeros.at[pl.ds(s * RPS, RPS)],
                      acc.at[pl.ds(s * RPS, RPS)])

      @pl.when(s == NS - 1)
      def _():
        pltpu.sync_copy(zeros.at[pl.ds(NS * RPS, RTL)],
                        acc.at[pl.ds(NS * RPS, RTL)])

      pltpu.async_copy(hn.at[si.at[0]], rows.at[0], s0)
      pltpu.async_copy(hn.at[si.at[1]], rows.at[1], s1)
      plsc.subcore_barrier()

      def pair(jo, carry):
        for b in range(2):
          j = jo * 2 + b
          pltpu.make_async_copy(hn.at[si.at[j]], rows.at[b], sems[b]).wait()
          pltpu.sync_copy(rows.at[b], acc.at[di.at[j]], add=True)
          pltpu.async_copy(hn.at[si.at[j + 2]], rows.at[b], sems[b])
        return carry
      lax.fori_loop(0, NCH // 2 - 1, pair, 0)

      for j, b in ((NCH - 2, 0), (NCH - 1, 1)):
        pltpu.make_async_copy(hn.at[si.at[j]], rows.at[b], sems[b]).wait()
        pltpu.sync_copy(rows.at[b], acc.at[di.at[j]], add=True)

      pltpu.async_copy(hn.at[st], rt, s0).wait()
      pltpu.sync_copy(rt, acc.at[dt], add=True)

      plsc.subcore_barrier()
      pltpu.sync_copy(acc.at[pl.ds(s * RPS, RPS)],
                      outs[p].at[c, pl.ds(s * RPS, RPS)])

      @pl.when(s == NS - 1)
      def _():
        pltpu.sync_copy(acc.at[pl.ds(NS * RPS, RTL)],
                        outs[p].at[c, pl.ds(NS * RPS, RTL)])

  return prop


def _prop(idx, hns, w):
  src_m, dst_m, src_t, dst_t = idx
  zeros = jnp.zeros((N, w), jnp.float32)
  outs = _make_prop(len(hns), w)(src_m, dst_m, src_t, dst_t, *hns, zeros)
  return outs if isinstance(outs, (list, tuple)) else [outs]


def _row_spec(*dims):
  nd = len(dims)
  if nd == 2:
    return pl.BlockSpec((BLK, dims[1]), lambda i: (i, 0))
  return pl.BlockSpec((dims[0], BLK, dims[2]), lambda i: (0, i, 0))


def _full_spec(shape):
  nd = len(shape)
  return pl.BlockSpec(shape, lambda i: (0,) * nd)


def _norm_hn(degp, x):
  def body(degp_r, x_r, norm_o, hn_o):
    d = degp_r[...]
    deg = d[0] + d[1]
    nrm = lax.rsqrt(jnp.maximum(deg, 1.0))
    norm_o[...] = nrm
    hn_o[...] = x_r[...] * nrm
  return pl.pallas_call(
      body, grid=(G,),
      in_specs=[_row_spec(2, N, 1), _row_spec(N, 128)],
      out_specs=[_row_spec(N, 1), _row_spec(N, 128)],
      out_shape=[jax.ShapeDtypeStruct((N, 1), jnp.float32),
                 jax.ShapeDtypeStruct((N, 128), jnp.float32)],
  )(degp, x)


def _combine_hop(parts, norm):
  def body(p0, p1, norm_r, ph_o, hn_o):
    nrm = norm_r[...]
    ph = jnp.concatenate([(p0[...][0] + p0[...][1]),
                          (p1[...][0] + p1[...][1])], axis=1) * nrm
    ph_o[...] = ph
    hn_o[...] = ph * nrm
  return pl.pallas_call(
      body, grid=(G,),
      in_specs=[_row_spec(2, N, 64)] * 2 + [_row_spec(N, 1)],
      out_specs=[_row_spec(N, 128)] * 2,
      out_shape=[jax.ShapeDtypeStruct((N, 128), jnp.float32)] * 2,
  )(*parts, norm)


def _layer1(x, px, p2, norm, W1, b1, wcat):
  def body(x_r, px_r, q0, q1, norm_r, w1_r, b1_r, wc_r,
           m2_o, h3a_o, h3b_o):
    nrm = norm_r[...]
    p2x = jnp.concatenate([(q0[...][0] + q0[...][1]),
                           (q1[...][0] + q1[...][1])], axis=1) * nrm
    w1 = w1_r[...]
    h1 = (jnp.dot(x_r[...], w1[0:128], preferred_element_type=jnp.float32)
          + jnp.dot(px_r[...], w1[128:256], preferred_element_type=jnp.float32)
          + jnp.dot(p2x, w1[256:384], preferred_element_type=jnp.float32)
          + b1_r[...])
    h1 = jnp.maximum(h1, 0.0)
    m2 = jnp.dot(h1, wc_r[...], preferred_element_type=jnp.float32)
    m2_o[...] = m2
    h3a_o[...] = m2[:, 40:80] * nrm
    h3b_o[...] = m2[:, 80:120] * nrm
  return pl.pallas_call(
      body, grid=(G,),
      in_specs=[_row_spec(N, 128), _row_spec(N, 128)]
      + [_row_spec(2, N, 64)] * 2
      + [_row_spec(N, 1), _full_spec((384, 128)), _full_spec((1, 128)),
         _full_spec((128, 120))],
      out_specs=[_row_spec(N, 120), _row_spec(N, 40), _row_spec(N, 40)],
      out_shape=[jax.ShapeDtypeStruct((N, 120), jnp.float32),
                 jax.ShapeDtypeStruct((N, 40), jnp.float32),
                 jax.ShapeDtypeStruct((N, 40), jnp.float32)],
  )(x, px, *p2, norm, W1, b1, wcat)


def _combine_l2hop1(parts, norm):
  def body(pa, pb, norm_r, zb_o, hn4_o):
    nrm = norm_r[...]
    pav = pa[...]
    pbv = pb[...]
    zb_o[...] = (pav[0] + pav[1]) * nrm
    hn4_o[...] = (pbv[0] + pbv[1]) * nrm * nrm
  return pl.pallas_call(
      body, grid=(G,),
      in_specs=[_row_spec(2, N, 40)] * 2 + [_row_spec(N, 1)],
      out_specs=[_row_spec(N, 40)] * 2,
      out_shape=[jax.ShapeDtypeStruct((N, 40), jnp.float32)] * 2,
  )(*parts, norm)


def _finalize(m2, zb, p4, norm, b2):
  def body(m2_r, zb_r, p4_r, norm_r, b2_r, out_o):
    i = pl.program_id(0)
    pv = p4_r[...]
    wv = (pv[0] + pv[1]) * norm_r[...]
    h2 = jnp.maximum(m2_r[...][:, 0:40] + zb_r[...] + wv + b2_r[...], 0.0)
    part = jnp.sum(h2, axis=0, keepdims=True) * (1.0 / N)

    @pl.when(i == 0)
    def _():
      out_o[...] = part

    @pl.when(i != 0)
    def _():
      out_o[...] += part

  return pl.pallas_call(
      body, grid=(G,),
      in_specs=[_row_spec(N, 120), _row_spec(N, 40), _row_spec(2, N, 40),
                _row_spec(N, 1), _full_spec((1, 40))],
      out_specs=pl.BlockSpec((1, 40), lambda i: (0, 0)),
      out_shape=jax.ShapeDtypeStruct((1, 40), jnp.float32),
  )(m2, zb, p4, norm, b2)


def kernel(in_feat, edge_index, W1, b1, W2, b2):
  src = edge_index[0].reshape(NW, EPW)
  dst = edge_index[1].reshape(NW, EPW)
  idx = (src[:, :NCH * CHUNK].reshape(NW, NCH, CHUNK),
         dst[:, :NCH * CHUNK].reshape(NW, NCH, CHUNK),
         src[:, NCH * CHUNK:],
         dst[:, NCH * CHUNK:])

  degp = _prop(idx, [jnp.ones((N, 1), jnp.float32)], 1)[0]
  norm, hn1 = _norm_hn(degp, in_feat)

  p1 = _prop(idx, [hn1[:, :64], hn1[:, 64:]], 64)
  px, hn2 = _combine_hop(p1, norm)
  p2 = _prop(idx, [hn2[:, :64], hn2[:, 64:]], 64)

  wcat = jnp.concatenate([W2[0:128], W2[128:256], W2[256:384]], axis=1)
  m2, h3a, h3b = _layer1(in_feat, px, p2, norm, W1, b1.reshape(1, 128), wcat)

  p3 = _prop(idx, [h3a, h3b], 40)
  zb, hn4 = _combine_l2hop1(p3, norm)
  p4 = _prop(idx, [hn4], 40)

  return _finalize(m2, zb, p4[0], norm, b2.reshape(1, 40))


# speedup vs baseline: 8.9854x; 1.0087x over previous
import functools

import jax
import jax.numpy as jnp
from jax import lax
from jax.experimental import pallas as pl
from jax.experimental.pallas import tpu as pltpu
from jax.experimental.pallas import tpu_sc as plsc

N = 10000
E = 320000
NC, NS = 2, 16
NW = NC * NS
EPW = E // NW
CHUNK = 128
NCH = EPW // CHUNK
TAIL = EPW - NCH * CHUNK
RPS = 624
RTL = N - RPS * NS
BLK = 1000
G = N // BLK


@functools.lru_cache(maxsize=None)
def _make_prop(nphase, w):
  mesh = plsc.VectorSubcoreMesh(
      core_axis_name="c", subcore_axis_name="s",
      num_cores=NC, num_subcores=NS)

  @functools.partial(
      pl.kernel,
      out_type=[jax.ShapeDtypeStruct((NC, N, w), jnp.float32)
                for _ in range(nphase)],
      mesh=mesh,
      compiler_params=pltpu.CompilerParams(use_tc_tiling_on_sc=False),
      scratch_types=[
          pltpu.VMEM((NCH, CHUNK), jnp.int32),
          pltpu.VMEM((NCH, CHUNK), jnp.int32),
          pltpu.VMEM((TAIL,), jnp.int32),
          pltpu.VMEM((TAIL,), jnp.int32),
          pltpu.VMEM((2, CHUNK, w), jnp.float32),
          pltpu.VMEM((TAIL, w), jnp.float32),
          pltpu.VMEM_SHARED((N, w), jnp.float32),
          pltpu.SemaphoreType.DMA,
          pltpu.SemaphoreType.DMA,
      ],
  )
  def prop(*args):
    src_m, dst_m, src_t, dst_t = args[0:4]
    hns = args[4:4 + nphase]
    zeros = args[4 + nphase]
    outs = args[5 + nphase:5 + 2 * nphase]
    si, di, st, dt, rows, rt, acc, s0, s1 = args[5 + 2 * nphase:]
    c = lax.axis_index("c")
    s = lax.axis_index("s")
    wk = c * NS + s
    sems = (s0, s1)

    pltpu.sync_copy(src_m.at[wk], si)
    pltpu.sync_copy(dst_m.at[wk], di)
    pltpu.sync_copy(src_t.at[wk], st)
    pltpu.sync_copy(dst_t.at[wk], dt)

    for p in range(nphase):
      hn = hns[p]
      pltpu.sync_copy(zeros.at[pl.ds(s * RPS, RPS)],
                      acc.at[pl.ds(s * RPS, RPS)])

      @pl.when(s == NS - 1)
      def _():
        pltpu.sync_copy(zeros.at[pl.ds(NS * RPS, RTL)],
                        acc.at[pl.ds(NS * RPS, RTL)])

      pltpu.async_copy(hn.at[si.at[0]], rows.at[0], s0)
      pltpu.async_copy(hn.at[si.at[1]], rows.at[1], s1)
      plsc.subcore_barrier()

      def pair(jo, carry):
        for b in range(2):
          j = jo * 2 + b
          pltpu.make_async_copy(hn.at[si.at[j]], rows.at[b], sems[b]).wait()
          pltpu.sync_copy(rows.at[b], acc.at[di.at[j]], add=True)
          pltpu.async_copy(hn.at[si.at[j + 2]], rows.at[b], sems[b])
        return carry
      lax.fori_loop(0, NCH // 2 - 1, pair, 0)

      for j, b in ((NCH - 2, 0), (NCH - 1, 1)):
        pltpu.make_async_copy(hn.at[si.at[j]], rows.at[b], sems[b]).wait()
        pltpu.sync_copy(rows.at[b], acc.at[di.at[j]], add=True)

      pltpu.async_copy(hn.at[st], rt, s0).wait()
      pltpu.sync_copy(rt, acc.at[dt], add=True)

      plsc.subcore_barrier()
      pltpu.sync_copy(acc.at[pl.ds(s * RPS, RPS)],
                      outs[p].at[c, pl.ds(s * RPS, RPS)])

      @pl.when(s == NS - 1)
      def _():
        pltpu.sync_copy(acc.at[pl.ds(NS * RPS, RTL)],
                        outs[p].at[c, pl.ds(NS * RPS, RTL)])

  return prop


def _prop(idx, hns, w):
  src_m, dst_m, src_t, dst_t = idx
  zeros = jnp.zeros((N, w), jnp.float32)
  outs = _make_prop(len(hns), w)(src_m, dst_m, src_t, dst_t, *hns, zeros)
  return outs if isinstance(outs, (list, tuple)) else [outs]


def _row_spec(*dims):
  nd = len(dims)
  if nd == 2:
    return pl.BlockSpec((BLK, dims[1]), lambda i: (i, 0))
  return pl.BlockSpec((dims[0], BLK, dims[2]), lambda i: (0, i, 0))


def _full_spec(shape):
  nd = len(shape)
  return pl.BlockSpec(shape, lambda i: (0,) * nd)


def _norm_hn(degp, x):
  def body(degp_r, x_r, norm_o, hn_o):
    d = degp_r[...]
    deg = d[0, :, 0:1] + d[1, :, 0:1]
    nrm = lax.rsqrt(jnp.maximum(deg, 1.0))
    norm_o[...] = nrm
    hn_o[...] = x_r[...] * nrm
  return pl.pallas_call(
      body, grid=(G,),
      in_specs=[_row_spec(2, N, 8), _row_spec(N, 128)],
      out_specs=[_row_spec(N, 1), _row_spec(N, 128)],
      out_shape=[jax.ShapeDtypeStruct((N, 1), jnp.float32),
                 jax.ShapeDtypeStruct((N, 128), jnp.float32)],
  )(degp, x)


def _combine_hop(parts, norm):
  def body(p0, p1, norm_r, ph_o, hn_o):
    nrm = norm_r[...]
    ph = jnp.concatenate([(p0[...][0] + p0[...][1]),
                          (p1[...][0] + p1[...][1])], axis=1) * nrm
    ph_o[...] = ph
    hn_o[...] = ph * nrm
  return pl.pallas_call(
      body, grid=(G,),
      in_specs=[_row_spec(2, N, 64)] * 2 + [_row_spec(N, 1)],
      out_specs=[_row_spec(N, 128)] * 2,
      out_shape=[jax.ShapeDtypeStruct((N, 128), jnp.float32)] * 2,
  )(*parts, norm)


def _layer1(x, px, p2, norm, W1, b1, wcat):
  def body(x_r, px_r, q0, q1, norm_r, w1_r, b1_r, wc_r,
           m2_o, h3a_o, h3b_o):
    nrm = norm_r[...]
    p2x = jnp.concatenate([(q0[...][0] + q0[...][1]),
                           (q1[...][0] + q1[...][1])], axis=1) * nrm
    w1 = w1_r[...]
    h1 = (jnp.dot(x_r[...], w1[0:128], preferred_element_type=jnp.float32)
          + jnp.dot(px_r[...], w1[128:256], preferred_element_type=jnp.float32)
          + jnp.dot(p2x, w1[256:384], preferred_element_type=jnp.float32)
          + b1_r[...])
    h1 = jnp.maximum(h1, 0.0)
    m2 = jnp.dot(h1, wc_r[...], preferred_element_type=jnp.float32)
    m2_o[...] = m2
    h3a_o[...] = m2[:, 40:80] * nrm
    h3b_o[...] = m2[:, 80:120] * nrm
  return pl.pallas_call(
      body, grid=(G,),
      in_specs=[_row_spec(N, 128), _row_spec(N, 128)]
      + [_row_spec(2, N, 64)] * 2
      + [_row_spec(N, 1), _full_spec((384, 128)), _full_spec((1, 128)),
         _full_spec((128, 120))],
      out_specs=[_row_spec(N, 120), _row_spec(N, 40), _row_spec(N, 40)],
      out_shape=[jax.ShapeDtypeStruct((N, 120), jnp.float32),
                 jax.ShapeDtypeStruct((N, 40), jnp.float32),
                 jax.ShapeDtypeStruct((N, 40), jnp.float32)],
  )(x, px, *p2, norm, W1, b1, wcat)


def _combine_l2hop1(parts, norm):
  def body(pa, pb, norm_r, zb_o, hn4_o):
    nrm = norm_r[...]
    pav = pa[...]
    pbv = pb[...]
    zb_o[...] = (pav[0] + pav[1]) * nrm
    hn4_o[...] = (pbv[0] + pbv[1]) * nrm * nrm
  return pl.pallas_call(
      body, grid=(G,),
      in_specs=[_row_spec(2, N, 40)] * 2 + [_row_spec(N, 1)],
      out_specs=[_row_spec(N, 40)] * 2,
      out_shape=[jax.ShapeDtypeStruct((N, 40), jnp.float32)] * 2,
  )(*parts, norm)


def _finalize(m2, zb, p4, norm, b2):
  def body(m2_r, zb_r, p4_r, norm_r, b2_r, out_o):
    i = pl.program_id(0)
    pv = p4_r[...]
    wv = (pv[0] + pv[1]) * norm_r[...]
    h2 = jnp.maximum(m2_r[...][:, 0:40] + zb_r[...] + wv + b2_r[...], 0.0)
    part = jnp.sum(h2, axis=0, keepdims=True) * (1.0 / N)

    @pl.when(i == 0)
    def _():
      out_o[...] = part

    @pl.when(i != 0)
    def _():
      out_o[...] += part

  return pl.pallas_call(
      body, grid=(G,),
      in_specs=[_row_spec(N, 120), _row_spec(N, 40), _row_spec(2, N, 40),
                _row_spec(N, 1), _full_spec((1, 40))],
      out_specs=pl.BlockSpec((1, 40), lambda i: (0, 0)),
      out_shape=jax.ShapeDtypeStruct((1, 40), jnp.float32),
  )(m2, zb, p4, norm, b2)


def kernel(in_feat, edge_index, W1, b1, W2, b2):
  src = edge_index[0].reshape(NW, EPW)
  dst = edge_index[1].reshape(NW, EPW)
  idx = (src[:, :NCH * CHUNK].reshape(NW, NCH, CHUNK),
         dst[:, :NCH * CHUNK].reshape(NW, NCH, CHUNK),
         src[:, NCH * CHUNK:],
         dst[:, NCH * CHUNK:])

  degp = _prop(idx, [jnp.ones((N, 8), jnp.float32)], 8)[0]
  norm, hn1 = _norm_hn(degp, in_feat)

  p1 = _prop(idx, [hn1[:, :64], hn1[:, 64:]], 64)
  px, hn2 = _combine_hop(p1, norm)
  p2 = _prop(idx, [hn2[:, :64], hn2[:, 64:]], 64)

  wcat = jnp.concatenate([W2[0:128], W2[128:256], W2[256:384]], axis=1)
  m2, h3a, h3b = _layer1(in_feat, px, p2, norm, W1, b1.reshape(1, 128), wcat)

  p3 = _prop(idx, [h3a, h3b], 40)
  zb, hn4 = _combine_l2hop1(p3, norm)
  p4 = _prop(idx, [hn4], 40)

  return _finalize(m2, zb, p4[0], norm, b2.reshape(1, 40))


# speedup vs baseline: 9.4264x; 1.0491x over previous
import functools

import jax
import jax.numpy as jnp
from jax import lax
from jax.experimental import pallas as pl
from jax.experimental.pallas import tpu as pltpu
from jax.experimental.pallas import tpu_sc as plsc

N = 10000
E = 320000
NC, NS = 2, 16
NW = NC * NS
CHUNK = 128

EPW = E // NW
NCH = EPW // CHUNK
TAIL = EPW - NCH * CHUNK

EPT = E // NS
NCH2 = EPT // CHUNK
TAIL2 = EPT - NCH2 * CHUNK

RPS = 624
RTL = N - RPS * NS
NBLK = 4
SBLK = RPS // NBLK

BLK = 1000
G = N // BLK

_SCPARAMS = pltpu.CompilerParams(use_tc_tiling_on_sc=False)


@functools.lru_cache(maxsize=None)
def _mesh():
  return plsc.VectorSubcoreMesh(
      core_axis_name="c", subcore_axis_name="s",
      num_cores=NC, num_subcores=NS)


def _edge_pass(table, si, di, st, dt, rows, rt, acc, s0, s1, nch):
  sems = (s0, s1)
  pltpu.async_copy(table.at[si.at[0]], rows.at[0], s0)
  pltpu.async_copy(table.at[si.at[1]], rows.at[1], s1)

  def pair(jo, carry):
    for b in range(2):
      j = jo * 2 + b
      pltpu.make_async_copy(table.at[si.at[j]], rows.at[b], sems[b]).wait()
      pltpu.sync_copy(rows.at[b], acc.at[di.at[j]], add=True)
      pltpu.async_copy(table.at[si.at[j + 2]], rows.at[b], sems[b])
    return carry
  lax.fori_loop(0, nch // 2 - 1, pair, 0)

  for j, b in ((nch - 2, 0), (nch - 1, 1)):
    pltpu.make_async_copy(table.at[si.at[j]], rows.at[b], sems[b]).wait()
    pltpu.sync_copy(rows.at[b], acc.at[di.at[j]], add=True)

  pltpu.async_copy(table.at[st], rt, s0).wait()
  pltpu.sync_copy(rt, acc.at[dt], add=True)


@functools.lru_cache(maxsize=None)
def _make_hop2():
  @functools.partial(
      pl.kernel,
      out_type=[jax.ShapeDtypeStruct((NC, N, 64), jnp.float32)
                for _ in range(3)],
      mesh=_mesh(),
      compiler_params=_SCPARAMS,
      scratch_types=[
          pltpu.VMEM((NCH2, CHUNK), jnp.int32),
          pltpu.VMEM((NCH2, CHUNK), jnp.int32),
          pltpu.VMEM((TAIL2,), jnp.int32),
          pltpu.VMEM((TAIL2,), jnp.int32),
          pltpu.VMEM((2, CHUNK, 64), jnp.float32),
          pltpu.VMEM((TAIL2, 64), jnp.float32),
          pltpu.VMEM((SBLK, 64), jnp.float32),
          pltpu.VMEM((SBLK, 64), jnp.float32),
          pltpu.VMEM_SHARED((N, 64), jnp.float32),
          pltpu.SemaphoreType.DMA,
          pltpu.SemaphoreType.DMA,
      ],
  )
  def hop2(src_m, dst_m, src_t, dst_t, hn1t, nrm64, zeros,
           pxh, hn2t, p2xh,
           si, di, st, dt, rows, rt, bb, nb, acc, s0, s1):
    c = lax.axis_index("c")
    s = lax.axis_index("s")
    base = s * RPS

    pltpu.sync_copy(src_m.at[s], si)
    pltpu.sync_copy(dst_m.at[s], di)
    pltpu.sync_copy(src_t.at[s], st)
    pltpu.sync_copy(dst_t.at[s], dt)
    pltpu.sync_copy(zeros.at[pl.ds(base, RPS)], acc.at[pl.ds(base, RPS)])

    @pl.when(s == NS - 1)
    def _():
      pltpu.sync_copy(zeros.at[pl.ds(NS * RPS, RTL)],
                      acc.at[pl.ds(NS * RPS, RTL)])

    plsc.subcore_barrier()
    _edge_pass(hn1t.at[c], si, di, st, dt, rows, rt, acc, s0, s1, NCH2)
    plsc.subcore_barrier()

    def mul_pass(nrows):
      def rowloop(r, carry):
        for k in range(4):
          bb[r, pl.ds(k * 16, 16)] = (bb[r, pl.ds(k * 16, 16)]
                                      * nb[r, pl.ds(k * 16, 16)])
        return carry
      lax.fori_loop(0, nrows, rowloop, 0)

    def scale_block(row0, nrows, outs):
      pltpu.sync_copy(acc.at[pl.ds(row0, nrows)], bb.at[pl.ds(0, nrows)])
      pltpu.sync_copy(nrm64.at[pl.ds(row0, nrows)], nb.at[pl.ds(0, nrows)])
      for out in outs:
        mul_pass(nrows)
        pltpu.sync_copy(bb.at[pl.ds(0, nrows)], out.at[c, pl.ds(row0, nrows)])

    for blk in range(NBLK):
      scale_block(base + blk * SBLK, SBLK, (pxh, hn2t))

    @pl.when(s == NS - 1)
    def _():
      scale_block(NS * RPS, RTL, (pxh, hn2t))

    pltpu.sync_copy(zeros.at[pl.ds(base, RPS)], acc.at[pl.ds(base, RPS)])

    @pl.when(s == NS - 1)
    def _():
      pltpu.sync_copy(zeros.at[pl.ds(NS * RPS, RTL)],
                      acc.at[pl.ds(NS * RPS, RTL)])

    plsc.subcore_barrier()
    _edge_pass(hn2t.at[c], si, di, st, dt, rows, rt, acc, s0, s1, NCH2)
    plsc.subcore_barrier()

    for blk in range(NBLK):
      scale_block(base + blk * SBLK, SBLK, (p2xh,))

    @pl.when(s == NS - 1)
    def _():
      scale_block(NS * RPS, RTL, (p2xh,))

  return hop2


def _hop2_call(idx2, hn1t, nrm64):
  src_m, dst_m, src_t, dst_t = idx2
  zeros = jnp.zeros((N, 64), jnp.float32)
  return _make_hop2()(src_m, dst_m, src_t, dst_t, hn1t, nrm64, zeros)


@functools.lru_cache(maxsize=None)
def _make_prop(nphase, w):
  @functools.partial(
      pl.kernel,
      out_type=[jax.ShapeDtypeStruct((NC, N, w), jnp.float32)
                for _ in range(nphase)],
      mesh=_mesh(),
      compiler_params=_SCPARAMS,
      scratch_types=[
          pltpu.VMEM((NCH, CHUNK), jnp.int32),
          pltpu.VMEM((NCH, CHUNK), jnp.int32),
          pltpu.VMEM((TAIL,), jnp.int32),
          pltpu.VMEM((TAIL,), jnp.int32),
          pltpu.VMEM((2, CHUNK, w), jnp.float32),
          pltpu.VMEM((TAIL, w), jnp.float32),
          pltpu.VMEM_SHARED((N, w), jnp.float32),
          pltpu.SemaphoreType.DMA,
          pltpu.SemaphoreType.DMA,
      ],
  )
  def prop(*args):
    src_m, dst_m, src_t, dst_t = args[0:4]
    hns = args[4:4 + nphase]
    zeros = args[4 + nphase]
    outs = args[5 + nphase:5 + 2 * nphase]
    si, di, st, dt, rows, rt, acc, s0, s1 = args[5 + 2 * nphase:]
    c = lax.axis_index("c")
    s = lax.axis_index("s")
    wk = c * NS + s

    pltpu.sync_copy(src_m.at[wk], si)
    pltpu.sync_copy(dst_m.at[wk], di)
    pltpu.sync_copy(src_t.at[wk], st)
    pltpu.sync_copy(dst_t.at[wk], dt)

    for p in range(nphase):
      pltpu.sync_copy(zeros.at[pl.ds(s * RPS, RPS)],
                      acc.at[pl.ds(s * RPS, RPS)])

      @pl.when(s == NS - 1)
      def _():
        pltpu.sync_copy(zeros.at[pl.ds(NS * RPS, RTL)],
                        acc.at[pl.ds(NS * RPS, RTL)])
      plsc.subcore_barrier()
      _edge_pass(hns[p], si, di, st, dt, rows, rt, acc, s0, s1, NCH)
      plsc.subcore_barrier()
      pltpu.sync_copy(acc.at[pl.ds(s * RPS, RPS)],
                      outs[p].at[c, pl.ds(s * RPS, RPS)])

      @pl.when(s == NS - 1)
      def _():
        pltpu.sync_copy(acc.at[pl.ds(NS * RPS, RTL)],
                        outs[p].at[c, pl.ds(NS * RPS, RTL)])

  return prop


def _prop(idx, hns, w):
  src_m, dst_m, src_t, dst_t = idx
  zeros = jnp.zeros((N, w), jnp.float32)
  outs = _make_prop(len(hns), w)(src_m, dst_m, src_t, dst_t, *hns, zeros)
  return outs if isinstance(outs, (list, tuple)) else [outs]


def _row_spec(*dims):
  nd = len(dims)
  if nd == 2:
    return pl.BlockSpec((BLK, dims[1]), lambda i: (i, 0))
  return pl.BlockSpec((dims[0], BLK, dims[2]), lambda i: (0, i, 0))


def _full_spec(shape):
  nd = len(shape)
  return pl.BlockSpec(shape, lambda i: (0,) * nd)


def _norm_hn(degp, x):
  def body(degp_r, x_r, norm_o, n64_o, hn_o):
    d = degp_r[...]
    deg = d[0, :, 0:1] + d[1, :, 0:1]
    nrm = lax.rsqrt(jnp.maximum(deg, 1.0))
    norm_o[...] = nrm
    n64_o[...] = jnp.broadcast_to(nrm, (nrm.shape[0], 64))
    hn = x_r[...] * nrm
    hn_o[...] = jnp.stack([hn[:, 0:64], hn[:, 64:128]], axis=0)
  return pl.pallas_call(
      body, grid=(G,),
      in_specs=[_row_spec(2, N, 8), _row_spec(N, 128)],
      out_specs=[_row_spec(N, 1), _row_spec(N, 64), _row_spec(2, N, 64)],
      out_shape=[jax.ShapeDtypeStruct((N, 1), jnp.float32),
                 jax.ShapeDtypeStruct((N, 64), jnp.float32),
                 jax.ShapeDtypeStruct((NC, N, 64), jnp.float32)],
  )(degp, x)


def _layer1(x, pxh, p2xh, norm, W1, b1, wcat):
  def body(x_r, px_r, p2x_r, norm_r, w1_r, b1_r, wc_r, m2_o, h3a_o, h3b_o):
    nrm = norm_r[...]
    pxv = px_r[...]
    p2xv = p2x_r[...]
    px = jnp.concatenate([pxv[0], pxv[1]], axis=1)
    p2x = jnp.concatenate([p2xv[0], p2xv[1]], axis=1)
    w1 = w1_r[...]
    h1 = (jnp.dot(x_r[...], w1[0:128], preferred_element_type=jnp.float32)
          + jnp.dot(px, w1[128:256], preferred_element_type=jnp.float32)
          + jnp.dot(p2x, w1[256:384], preferred_element_type=jnp.float32)
          + b1_r[...])
    h1 = jnp.maximum(h1, 0.0)
    m2 = jnp.dot(h1, wc_r[...], preferred_element_type=jnp.float32)
    m2_o[...] = m2
    h3a_o[...] = m2[:, 40:80] * nrm
    h3b_o[...] = m2[:, 80:120] * nrm
  return pl.pallas_call(
      body, grid=(G,),
      in_specs=[_row_spec(N, 128), _row_spec(2, N, 64), _row_spec(2, N, 64),
                _row_spec(N, 1), _full_spec((384, 128)), _full_spec((1, 128)),
                _full_spec((128, 120))],
      out_specs=[_row_spec(N, 120), _row_spec(N, 40), _row_spec(N, 40)],
      out_shape=[jax.ShapeDtypeStruct((N, 120), jnp.float32),
                 jax.ShapeDtypeStruct((N, 40), jnp.float32),
                 jax.ShapeDtypeStruct((N, 40), jnp.float32)],
  )(x, pxh, p2xh, norm, W1, b1, wcat)


def _combine_l2hop1(parts, norm):
  def body(pa, pb, norm_r, zb_o, hn4_o):
    nrm = norm_r[...]
    pav = pa[...]
    pbv = pb[...]
    zb_o[...] = (pav[0] + pav[1]) * nrm
    hn4_o[...] = (pbv[0] + pbv[1]) * nrm * nrm
  return pl.pallas_call(
      body, grid=(G,),
      in_specs=[_row_spec(2, N, 40)] * 2 + [_row_spec(N, 1)],
      out_specs=[_row_spec(N, 40)] * 2,
      out_shape=[jax.ShapeDtypeStruct((N, 40), jnp.float32)] * 2,
  )(*parts, norm)


def _finalize(m2, zb, p4, norm, b2):
  def body(m2_r, zb_r, p4_r, norm_r, b2_r, out_o):
    i = pl.program_id(0)
    pv = p4_r[...]
    wv = (pv[0] + pv[1]) * norm_r[...]
    h2 = jnp.maximum(m2_r[...][:, 0:40] + zb_r[...] + wv + b2_r[...], 0.0)
    part = jnp.sum(h2, axis=0, keepdims=True) * (1.0 / N)

    @pl.when(i == 0)
    def _():
      out_o[...] = part

    @pl.when(i != 0)
    def _():
      out_o[...] += part

  return pl.pallas_call(
      body, grid=(G,),
      in_specs=[_row_spec(N, 120), _row_spec(N, 40), _row_spec(2, N, 40),
                _row_spec(N, 1), _full_spec((1, 40))],
      out_specs=pl.BlockSpec((1, 40), lambda i: (0, 0)),
      out_shape=jax.ShapeDtypeStruct((1, 40), jnp.float32),
  )(m2, zb, p4, norm, b2)


def kernel(in_feat, edge_index, W1, b1, W2, b2):
  srcf = edge_index[0]
  dstf = edge_index[1]

  src = srcf.reshape(NW, EPW)
  dst = dstf.reshape(NW, EPW)
  idx = (src[:, :NCH * CHUNK].reshape(NW, NCH, CHUNK),
         dst[:, :NCH * CHUNK].reshape(NW, NCH, CHUNK),
         src[:, NCH * CHUNK:],
         dst[:, NCH * CHUNK:])

  src2 = srcf.reshape(NS, EPT)
  dst2 = dstf.reshape(NS, EPT)
  idx2 = (src2[:, :NCH2 * CHUNK].reshape(NS, NCH2, CHUNK),
          dst2[:, :NCH2 * CHUNK].reshape(NS, NCH2, CHUNK),
          src2[:, NCH2 * CHUNK:],
          dst2[:, NCH2 * CHUNK:])

  degp = _prop(idx, [jnp.ones((N, 8), jnp.float32)], 8)[0]
  norm, nrm64, hn1t = _norm_hn(degp, in_feat)

  pxh, _, p2xh = _hop2_call(idx2, hn1t, nrm64)

  wcat = jnp.concatenate([W2[0:128], W2[128:256], W2[256:384]], axis=1)
  m2, h3a, h3b = _layer1(in_feat, pxh, p2xh, norm, W1,
                         b1.reshape(1, 128), wcat)

  p3 = _prop(idx, [h3a, h3b], 40)
  zb, hn4 = _combine_l2hop1(p3, norm)
  p4 = _prop(idx, [hn4], 40)

  return _finalize(m2, zb, p4[0], norm, b2.reshape(1, 40))


# speedup vs baseline: 11.0600x; 1.1733x over previous
import functools

import jax
import jax.numpy as jnp
from jax import lax
from jax.experimental import pallas as pl
from jax.experimental.pallas import tpu as pltpu
from jax.experimental.pallas import tpu_sc as plsc

N = 10000
E = 320000
NC, NS = 2, 16
NW = NC * NS
CHUNK = 128

EPW = E // NW
NCH = EPW // CHUNK
TAIL = EPW - NCH * CHUNK

EPT = E // NS
NCH2 = EPT // CHUNK
TAIL2 = EPT - NCH2 * CHUNK

RPS = 624
RTL = N - RPS * NS
NBLK = 4
SBLK = RPS // NBLK

BLK = 1000
G = N // BLK

_SCPARAMS = pltpu.CompilerParams(use_tc_tiling_on_sc=False)


@functools.lru_cache(maxsize=None)
def _mesh():
  return plsc.VectorSubcoreMesh(
      core_axis_name="c", subcore_axis_name="s",
      num_cores=NC, num_subcores=NS)


def _edge_pass(table, si, di, st, dt, rows, rt, acc, s0, s1, nch):
  sems = (s0, s1)
  pltpu.async_copy(table.at[si.at[0]], rows.at[0], s0)
  pltpu.async_copy(table.at[si.at[1]], rows.at[1], s1)

  def pair(jo, carry):
    for b in range(2):
      j = jo * 2 + b
      pltpu.make_async_copy(table.at[si.at[j]], rows.at[b], sems[b]).wait()
      pltpu.sync_copy(rows.at[b], acc.at[di.at[j]], add=True)
      pltpu.async_copy(table.at[si.at[j + 2]], rows.at[b], sems[b])
    return carry
  lax.fori_loop(0, nch // 2 - 1, pair, 0)

  for j, b in ((nch - 2, 0), (nch - 1, 1)):
    pltpu.make_async_copy(table.at[si.at[j]], rows.at[b], sems[b]).wait()
    pltpu.sync_copy(rows.at[b], acc.at[di.at[j]], add=True)

  pltpu.async_copy(table.at[st], rt, s0).wait()
  pltpu.sync_copy(rt, acc.at[dt], add=True)


@functools.lru_cache(maxsize=None)
def _make_hop2():
  @functools.partial(
      pl.kernel,
      out_type=[jax.ShapeDtypeStruct((NC, N, 64), jnp.float32)
                for _ in range(3)],
      mesh=_mesh(),
      compiler_params=_SCPARAMS,
      scratch_types=[
          pltpu.VMEM((NCH2, CHUNK), jnp.int32),
          pltpu.VMEM((NCH2, CHUNK), jnp.int32),
          pltpu.VMEM((TAIL2,), jnp.int32),
          pltpu.VMEM((TAIL2,), jnp.int32),
          pltpu.VMEM((2, CHUNK, 64), jnp.float32),
          pltpu.VMEM((TAIL2, 64), jnp.float32),
          pltpu.VMEM((SBLK, 64), jnp.float32),
          pltpu.VMEM((SBLK, 64), jnp.float32),
          pltpu.VMEM_SHARED((N, 64), jnp.float32),
          pltpu.SemaphoreType.DMA,
          pltpu.SemaphoreType.DMA,
      ],
  )
  def hop2(src_m, dst_m, src_t, dst_t, hn1t, nrm64, zeros,
           pxh, hn2t, p2xh,
           si, di, st, dt, rows, rt, bb, nb, acc, s0, s1):
    c = lax.axis_index("c")
    s = lax.axis_index("s")
    base = s * RPS

    pltpu.sync_copy(src_m.at[s], si)
    pltpu.sync_copy(dst_m.at[s], di)
    pltpu.sync_copy(src_t.at[s], st)
    pltpu.sync_copy(dst_t.at[s], dt)
    pltpu.sync_copy(zeros.at[pl.ds(base, RPS)], acc.at[pl.ds(base, RPS)])

    @pl.when(s == NS - 1)
    def _():
      pltpu.sync_copy(zeros.at[pl.ds(NS * RPS, RTL)],
                      acc.at[pl.ds(NS * RPS, RTL)])

    plsc.subcore_barrier()
    _edge_pass(hn1t.at[c], si, di, st, dt, rows, rt, acc, s0, s1, NCH2)
    plsc.subcore_barrier()

    def mul_pass(nrows):
      def rowloop(r, carry):
        for k in range(4):
          bb[r, pl.ds(k * 16, 16)] = (bb[r, pl.ds(k * 16, 16)]
                                      * nb[r, pl.ds(k * 16, 16)])
        return carry
      lax.fori_loop(0, nrows, rowloop, 0)

    def scale_block(row0, nrows, outs):
      pltpu.sync_copy(acc.at[pl.ds(row0, nrows)], bb.at[pl.ds(0, nrows)])
      pltpu.sync_copy(nrm64.at[pl.ds(row0, nrows)], nb.at[pl.ds(0, nrows)])
      for out in outs:
        mul_pass(nrows)
        pltpu.sync_copy(bb.at[pl.ds(0, nrows)], out.at[c, pl.ds(row0, nrows)])

    for blk in range(NBLK):
      scale_block(base + blk * SBLK, SBLK, (pxh, hn2t))

    @pl.when(s == NS - 1)
    def _():
      scale_block(NS * RPS, RTL, (pxh, hn2t))

    pltpu.sync_copy(zeros.at[pl.ds(base, RPS)], acc.at[pl.ds(base, RPS)])

    @pl.when(s == NS - 1)
    def _():
      pltpu.sync_copy(zeros.at[pl.ds(NS * RPS, RTL)],
                      acc.at[pl.ds(NS * RPS, RTL)])

    plsc.subcore_barrier()
    _edge_pass(hn2t.at[c], si, di, st, dt, rows, rt, acc, s0, s1, NCH2)
    plsc.subcore_barrier()

    for blk in range(NBLK):
      scale_block(base + blk * SBLK, SBLK, (p2xh,))

    @pl.when(s == NS - 1)
    def _():
      scale_block(NS * RPS, RTL, (p2xh,))

  return hop2


def _hop2_call(idx2, hn1t, nrm64):
  src_m, dst_m, src_t, dst_t = idx2
  zeros = jnp.zeros((N, 64), jnp.float32)
  return _make_hop2()(src_m, dst_m, src_t, dst_t, hn1t, nrm64, zeros)


@functools.lru_cache(maxsize=None)
def _make_deg():
  @functools.partial(
      pl.kernel,
      out_type=jax.ShapeDtypeStruct((NW, N), jnp.float32),
      mesh=_mesh(),
      compiler_params=pltpu.CompilerParams(
          use_tc_tiling_on_sc=False, needs_layout_passes=False),
      scratch_types=[
          pltpu.VMEM((NCH, CHUNK), jnp.int32),
          pltpu.VMEM((TAIL,), jnp.int32),
          pltpu.VMEM((N,), jnp.float32),
      ],
  )
  def dk(dst_m, dst_t, out, di, dt, dacc):
    c = lax.axis_index("c")
    s = lax.axis_index("s")
    wk = c * NS + s
    pltpu.sync_copy(dst_m.at[wk], di)
    pltpu.sync_copy(dst_t.at[wk], dt)

    def zloop(i, carry):
      dacc[pl.ds(i * 16, 16)] = jnp.zeros((16,), jnp.float32)
      return carry
    lax.fori_loop(0, N // 16, zloop, 0)

    ones = jnp.full((16,), 1.0, jnp.float32)
    vpc = CHUNK // 16

    def dloop(i, carry):
      j = i // vpc
      k = i % vpc
      dv = di[j, pl.ds(k * 16, 16)]
      plsc.addupdate_scatter(dacc, [dv], ones)
      return carry
    lax.fori_loop(0, NCH * vpc, dloop, 0)
    plsc.addupdate_scatter(dacc, [dt[pl.ds(0, 16)]], ones)
    pltpu.sync_copy(dacc, out.at[wk])

  return dk


def _deg_call(idx):
  return _make_deg()(idx[1], idx[3])


@functools.lru_cache(maxsize=None)
def _make_prop(nphase, w):
  @functools.partial(
      pl.kernel,
      out_type=[jax.ShapeDtypeStruct((NC, N, w), jnp.float32)
                for _ in range(nphase)],
      mesh=_mesh(),
      compiler_params=_SCPARAMS,
      scratch_types=[
          pltpu.VMEM((NCH, CHUNK), jnp.int32),
          pltpu.VMEM((NCH, CHUNK), jnp.int32),
          pltpu.VMEM((TAIL,), jnp.int32),
          pltpu.VMEM((TAIL,), jnp.int32),
          pltpu.VMEM((2, CHUNK, w), jnp.float32),
          pltpu.VMEM((TAIL, w), jnp.float32),
          pltpu.VMEM_SHARED((N, w), jnp.float32),
          pltpu.SemaphoreType.DMA,
          pltpu.SemaphoreType.DMA,
      ],
  )
  def prop(*args):
    src_m, dst_m, src_t, dst_t = args[0:4]
    hns = args[4:4 + nphase]
    zeros = args[4 + nphase]
    outs = args[5 + nphase:5 + 2 * nphase]
    si, di, st, dt, rows, rt, acc, s0, s1 = args[5 + 2 * nphase:]
    c = lax.axis_index("c")
    s = lax.axis_index("s")
    wk = c * NS + s

    pltpu.sync_copy(src_m.at[wk], si)
    pltpu.sync_copy(dst_m.at[wk], di)
    pltpu.sync_copy(src_t.at[wk], st)
    pltpu.sync_copy(dst_t.at[wk], dt)

    for p in range(nphase):
      pltpu.sync_copy(zeros.at[pl.ds(s * RPS, RPS)],
                      acc.at[pl.ds(s * RPS, RPS)])

      @pl.when(s == NS - 1)
      def _():
        pltpu.sync_copy(zeros.at[pl.ds(NS * RPS, RTL)],
                        acc.at[pl.ds(NS * RPS, RTL)])
      plsc.subcore_barrier()
      _edge_pass(hns[p], si, di, st, dt, rows, rt, acc, s0, s1, NCH)
      plsc.subcore_barrier()
      pltpu.sync_copy(acc.at[pl.ds(s * RPS, RPS)],
                      outs[p].at[c, pl.ds(s * RPS, RPS)])

      @pl.when(s == NS - 1)
      def _():
        pltpu.sync_copy(acc.at[pl.ds(NS * RPS, RTL)],
                        outs[p].at[c, pl.ds(NS * RPS, RTL)])

  return prop


def _prop(idx, hns, w):
  src_m, dst_m, src_t, dst_t = idx
  zeros = jnp.zeros((N, w), jnp.float32)
  outs = _make_prop(len(hns), w)(src_m, dst_m, src_t, dst_t, *hns, zeros)
  return outs if isinstance(outs, (list, tuple)) else [outs]


def _row_spec(*dims):
  nd = len(dims)
  if nd == 2:
    return pl.BlockSpec((BLK, dims[1]), lambda i: (i, 0))
  return pl.BlockSpec((dims[0], BLK, dims[2]), lambda i: (0, i, 0))


def _full_spec(shape):
  nd = len(shape)
  return pl.BlockSpec(shape, lambda i: (0,) * nd)


def _norm_hn(degp, x):
  def body(degp_r, x_r, norm_o, n64_o, hn_o):
    d = degp_r[...]
    deg = jnp.sum(d, axis=0)[:, None]
    nrm = lax.rsqrt(jnp.maximum(deg, 1.0))
    norm_o[...] = nrm
    n64_o[...] = jnp.broadcast_to(nrm, (nrm.shape[0], 64))
    hn = x_r[...] * nrm
    hn_o[...] = jnp.stack([hn[:, 0:64], hn[:, 64:128]], axis=0)
  return pl.pallas_call(
      body, grid=(1,),
      in_specs=[_full_spec((NW, N)), _full_spec((N, 128))],
      out_specs=[_full_spec((N, 1)), _full_spec((N, 64)),
                 _full_spec((NC, N, 64))],
      out_shape=[jax.ShapeDtypeStruct((N, 1), jnp.float32),
                 jax.ShapeDtypeStruct((N, 64), jnp.float32),
                 jax.ShapeDtypeStruct((NC, N, 64), jnp.float32)],
  )(degp, x)


def _layer1(x, pxh, p2xh, norm, W1, b1, wcat):
  def body(x_r, px_r, p2x_r, norm_r, w1_r, b1_r, wc_r, m2_o, h3_o):
    nrm = norm_r[...]
    pxv = px_r[...]
    p2xv = p2x_r[...]
    px = jnp.concatenate([pxv[0], pxv[1]], axis=1)
    p2x = jnp.concatenate([p2xv[0], p2xv[1]], axis=1)
    w1 = w1_r[...]
    h1 = (jnp.dot(x_r[...], w1[0:128], preferred_element_type=jnp.float32)
          + jnp.dot(px, w1[128:256], preferred_element_type=jnp.float32)
          + jnp.dot(p2x, w1[256:384], preferred_element_type=jnp.float32)
          + b1_r[...])
    h1 = jnp.maximum(h1, 0.0)
    m2 = jnp.dot(h1, wc_r[...], preferred_element_type=jnp.float32)
    m2_o[...] = m2
    h3_o[...] = m2[:, 40:120] * nrm
  return pl.pallas_call(
      body, grid=(G,),
      in_specs=[_row_spec(N, 128), _row_spec(2, N, 64), _row_spec(2, N, 64),
                _row_spec(N, 1), _full_spec((384, 128)), _full_spec((1, 128)),
                _full_spec((128, 120))],
      out_specs=[_row_spec(N, 120), _row_spec(N, 80)],
      out_shape=[jax.ShapeDtypeStruct((N, 120), jnp.float32),
                 jax.ShapeDtypeStruct((N, 80), jnp.float32)],
  )(x, pxh, p2xh, norm, W1, b1, wcat)


def _combine_l2hop1(parts, norm):
  def body(pa, norm_r, zb_o, hn4_o):
    nrm = norm_r[...]
    sv = pa[...][0] + pa[...][1]
    zb_o[...] = sv[:, 0:40] * nrm
    hn4_o[...] = sv[:, 40:80] * nrm * nrm
  return pl.pallas_call(
      body, grid=(G,),
      in_specs=[_row_spec(2, N, 80), _row_spec(N, 1)],
      out_specs=[_row_spec(N, 40)] * 2,
      out_shape=[jax.ShapeDtypeStruct((N, 40), jnp.float32)] * 2,
  )(parts, norm)


def _finalize(m2, zb, p4, norm, b2):
  def body(m2_r, zb_r, p4_r, norm_r, b2_r, out_o):
    i = pl.program_id(0)
    pv = p4_r[...]
    wv = (pv[0] + pv[1]) * norm_r[...]
    h2 = jnp.maximum(m2_r[...][:, 0:40] + zb_r[...] + wv + b2_r[...], 0.0)
    part = jnp.sum(h2, axis=0, keepdims=True) * (1.0 / N)

    @pl.when(i == 0)
    def _():
      out_o[...] = part

    @pl.when(i != 0)
    def _():
      out_o[...] += part

  return pl.pallas_call(
      body, grid=(G,),
      in_specs=[_row_spec(N, 120), _row_spec(N, 40), _row_spec(2, N, 40),
                _row_spec(N, 1), _full_spec((1, 40))],
      out_specs=pl.BlockSpec((1, 40), lambda i: (0, 0)),
      out_shape=jax.ShapeDtypeStruct((1, 40), jnp.float32),
  )(m2, zb, p4, norm, b2)


def kernel(in_feat, edge_index, W1, b1, W2, b2):
  srcf = edge_index[0]
  dstf = edge_index[1]

  src = srcf.reshape(NW, EPW)
  dst = dstf.reshape(NW, EPW)
  idx = (src[:, :NCH * CHUNK].reshape(NW, NCH, CHUNK),
         dst[:, :NCH * CHUNK].reshape(NW, NCH, CHUNK),
         src[:, NCH * CHUNK:],
         dst[:, NCH * CHUNK:])

  src2 = srcf.reshape(NS, EPT)
  dst2 = dstf.reshape(NS, EPT)
  idx2 = (src2[:, :NCH2 * CHUNK].reshape(NS, NCH2, CHUNK),
          dst2[:, :NCH2 * CHUNK].reshape(NS, NCH2, CHUNK),
          src2[:, NCH2 * CHUNK:],
          dst2[:, NCH2 * CHUNK:])

  degp = _deg_call(idx)
  norm, nrm64, hn1t = _norm_hn(degp, in_feat)

  pxh, _, p2xh = _hop2_call(idx2, hn1t, nrm64)

  wcat = jnp.concatenate([W2[0:128], W2[128:256], W2[256:384]], axis=1)
  m2, h3 = _layer1(in_feat, pxh, p2xh, norm, W1,
                   b1.reshape(1, 128), wcat)

  p3 = _prop(idx, [h3], 80)[0]
  zb, hn4 = _combine_l2hop1(p3, norm)
  p4 = _prop(idx, [hn4], 40)

  return _finalize(m2, zb, p4[0], norm, b2.reshape(1, 40))
